# baseline (speedup 1.0000x reference)
import jax
import jax.numpy as jnp
from jax import lax
from jax.experimental import pallas as pl
from jax.experimental.pallas import tpu as pltpu
from jax.experimental.pallas import tpu_sc as plsc

NC = 2
NS = 16
L = 16
NW = NC * NS

BATCH = 16384
D = 64
W = 2 * D
B_PER_W = BATCH // NW
CHUNK = 128
NCHUNK = B_PER_W // CHUNK
GROUPS = CHUNK // L
JBLK = 16


def _softplus(x):
    ax = jnp.abs(x)
    z = jnp.exp(-ax)
    t = z / (2.0 + z)
    t2 = t * t
    log1p = t * (2.0 + t2 * (2.0 / 3.0 + t2 * (0.4 + t2 * (2.0 / 7.0))))
    return jnp.maximum(x, 0.0) + log1p


def _bpr_body(uq_hbm, iq_hbm, jq_hbm, ur_hbm, ir_hbm, jr_hbm,
              eu_hbm, ei_hbm,
              out_sp_hbm, out_sq_hbm,
              uqv, iqv, jqv, urv, irv, jrv, ubuf, pbuf, nbuf,
              sp_stage, sq_stage, sem0, sem1):
    wid = lax.axis_index("s") * NC + lax.axis_index("c")
    sems = (sem0, sem1)

    pltpu.sync_copy(uq_hbm.at[wid], uqv)
    pltpu.sync_copy(iq_hbm.at[wid], iqv)
    pltpu.sync_copy(jq_hbm.at[wid], jqv)
    pltpu.sync_copy(ur_hbm.at[wid], urv)
    pltpu.sync_copy(ir_hbm.at[wid], irv)
    pltpu.sync_copy(jr_hbm.at[wid], jrv)

    def fire(c):
        s = c % 2
        pltpu.async_copy(eu_hbm.at[uqv.at[c]], ubuf.at[s], sems[s])
        pltpu.async_copy(ei_hbm.at[iqv.at[c]], pbuf.at[s], sems[s])
        pltpu.async_copy(ei_hbm.at[jqv.at[c]], nbuf.at[s], sems[s])

    def drain(c):
        s = c % 2
        pltpu.make_async_copy(eu_hbm.at[uqv.at[c]], ubuf.at[s], sems[s]).wait()
        pltpu.make_async_copy(ei_hbm.at[iqv.at[c]], pbuf.at[s], sems[s]).wait()
        pltpu.make_async_copy(ei_hbm.at[jqv.at[c]], nbuf.at[s], sems[s]).wait()

    iota = lax.iota(jnp.int32, L)
    zero = jnp.zeros((L,), jnp.float32)

    def compute_chunk(c, ub, pb, nb, carry):
        def group(g, carry):
            sp_acc, qu0, qu1, qp0, qp1, qn0, qn1 = carry
            row = g * L + iota
            base = g * L
            uoff = jnp.left_shift(jnp.bitwise_and(urv[c, pl.ds(base, L)], 1), 6)
            ioff = jnp.left_shift(jnp.bitwise_and(irv[c, pl.ds(base, L)], 1), 6)
            joff = jnp.left_shift(jnp.bitwise_and(jrv[c, pl.ds(base, L)], 1), 6)

            def jblock(jb, inner):
                d = list(inner[0:4])
                q_u = list(inner[4:6])
                q_p = list(inner[6:8])
                q_n = list(inner[8:10])
                jbase = iota + jb * JBLK
                for jj in range(JBLK):
                    col = jnp.bitwise_and(jbase + jj, D - 1)
                    u = plsc.load_gather(ub, [row, uoff + col])
                    p = plsc.load_gather(pb, [row, ioff + col])
                    n = plsc.load_gather(nb, [row, joff + col])
                    d[jj % 4] = d[jj % 4] + u * (n - p)
                    q_u[jj % 2] = q_u[jj % 2] + u * u
                    q_p[jj % 2] = q_p[jj % 2] + p * p
                    q_n[jj % 2] = q_n[jj % 2] + n * n
                return (*d, *q_u, *q_p, *q_n)

            inner = (zero, zero, zero, zero, qu0, qu1, qp0, qp1, qn0, qn1)
            inner = lax.fori_loop(0, D // JBLK, jblock, inner)
            d0, d1, d2, d3, qu0, qu1, qp0, qp1, qn0, qn1 = inner
            dt = (d0 + d1) + (d2 + d3)
            sp_acc = sp_acc + _softplus(dt)
            return (sp_acc, qu0, qu1, qp0, qp1, qn0, qn1)

        return lax.fori_loop(0, GROUPS, group, carry)

    carry = (zero,) * 7
    fire(0)
    for c in range(NCHUNK):
        if c + 1 < NCHUNK:
            fire(c + 1)
        drain(c)
        s = c % 2
        carry = compute_chunk(c, ubuf.at[s], pbuf.at[s], nbuf.at[s], carry)

    sp_acc, qu0, qu1, qp0, qp1, qn0, qn1 = carry
    sq = (qu0 + qu1) + (qp0 + qp1) + (qn0 + qn1)
    sp_stage[...] = sp_acc
    sq_stage[...] = sq
    pltpu.sync_copy(sp_stage, out_sp_hbm.at[wid])
    pltpu.sync_copy(sq_stage, out_sq_hbm.at[wid])


@jax.jit
def _bpr_call(uq, iq, jq, ur, ir, jr, embed_user, embed_item):
    eu_p = embed_user.reshape(500000, W)
    ei_p = embed_item.reshape(500000, W)
    mesh = plsc.VectorSubcoreMesh(core_axis_name="c", subcore_axis_name="s")
    f = pl.kernel(
        _bpr_body,
        out_type=[
            jax.ShapeDtypeStruct((NW, L), jnp.float32),
            jax.ShapeDtypeStruct((NW, L), jnp.float32),
        ],
        mesh=mesh,
        compiler_params=pltpu.CompilerParams(needs_layout_passes=False),
        scratch_types=[
            pltpu.VMEM((NCHUNK, CHUNK), jnp.int32),
            pltpu.VMEM((NCHUNK, CHUNK), jnp.int32),
            pltpu.VMEM((NCHUNK, CHUNK), jnp.int32),
            pltpu.VMEM((NCHUNK, CHUNK), jnp.int32),
            pltpu.VMEM((NCHUNK, CHUNK), jnp.int32),
            pltpu.VMEM((NCHUNK, CHUNK), jnp.int32),
            pltpu.VMEM((2, CHUNK, W), jnp.float32),
            pltpu.VMEM((2, CHUNK, W), jnp.float32),
            pltpu.VMEM((2, CHUNK, W), jnp.float32),
            pltpu.VMEM((L,), jnp.float32),
            pltpu.VMEM((L,), jnp.float32),
            pltpu.SemaphoreType.DMA,
            pltpu.SemaphoreType.DMA,
        ],
    )
    sp_part, sq_part = f(uq, iq, jq, ur, ir, jr, eu_p, ei_p)
    inv_b = 1.0 / BATCH
    loss = jnp.sum(sp_part) * inv_b
    reg = 0.5 * jnp.sum(sq_part) * inv_b
    return loss, reg


def kernel(user, item_i, item_j, embed_user, embed_item):
    user_i32 = user.astype(jnp.int32)
    item_i_i32 = item_i.astype(jnp.int32)
    item_j_i32 = item_j.astype(jnp.int32)
    shp = (NW, NCHUNK, CHUNK)
    uq = jnp.right_shift(user_i32, 1).reshape(shp)
    iq = jnp.right_shift(item_i_i32, 1).reshape(shp)
    jq = jnp.right_shift(item_j_i32, 1).reshape(shp)
    ur = user_i32.reshape(shp)
    ir = item_i_i32.reshape(shp)
    jr = item_j_i32.reshape(shp)
    return _bpr_call(uq, iq, jq, ur, ir, jr, embed_user, embed_item)
